# baseline (speedup 1.0000x reference)
import functools

import jax
import jax.numpy as jnp
from jax import lax
from jax.experimental import pallas as pl
from jax.experimental.pallas import tpu as pltpu
from jax.experimental.pallas import tpu_sc as plsc

N_NODES = 10000
D_FEAT = 128
HIDDEN = 64
N_CLASSES = 47
C_PAD = 48

NC, NS = 2, 16
NW = NC * NS
CHUNK = 128
ABUF = 4
N_PAD = 10240
RPT = N_PAD // NS


def _sc_mesh():
    return plsc.VectorSubcoreMesh(core_axis_name="c", subcore_axis_name="s")


_SC_PARAMS = dict(
    compiler_params=pltpu.CompilerParams(use_tc_tiling_on_sc=False,
                                         needs_layout_passes=False),
)


def _make_deg_kernel(per_s):
    nch = per_s // 2

    @functools.partial(
        pl.kernel,
        out_type=jax.ShapeDtypeStruct((NW, N_PAD), jnp.float32),
        mesh=_sc_mesh(),
        scratch_types=[
            pltpu.VMEM((nch, CHUNK), jnp.int32),
            pltpu.VMEM((N_PAD,), jnp.float32),
        ],
        **_SC_PARAMS,
    )
    def k(zeros_hbm, dst_hbm, out_hbm, dst_v, hist):
        c = lax.axis_index("c")
        s = lax.axis_index("s")
        wid = s * NC + c
        pltpu.sync_copy(dst_hbm.at[pl.ds(wid * nch, nch)], dst_v)
        pltpu.sync_copy(zeros_hbm, hist)
        ones = jnp.ones((16,), jnp.float32)

        def chunk(j, carry):
            for kk in range(CHUNK // 16):
                idx = dst_v[j, pl.ds(kk * 16, 16)]
                plsc.addupdate_scatter(hist, [idx], ones)
            return carry

        lax.fori_loop(0, nch, chunk, 0)
        pltpu.sync_copy(hist, out_hbm.at[wid])

    return k


def _make_agg_kernel(d, nch):
    assert nch % (4 * ABUF) == 0

    @functools.partial(
        pl.kernel,
        out_type=jax.ShapeDtypeStruct((NC * N_PAD, d), jnp.float32),
        mesh=_sc_mesh(),
        scratch_types=[
            pltpu.VMEM((2, ABUF, CHUNK), jnp.int32),
            pltpu.VMEM((4, ABUF, CHUNK), jnp.int32),
            pltpu.VMEM((2, ABUF, CHUNK, d), jnp.float32),
            pltpu.VMEM_SHARED((N_PAD, d), jnp.float32),
            pltpu.SemaphoreType.DMA,
            pltpu.SemaphoreType.DMA,
            pltpu.SemaphoreType.DMA,
            pltpu.SemaphoreType.DMA,
        ],
        **_SC_PARAMS,
    )
    def k(table_hbm, src_hbm, dst_hbm, out_hbm,
          srcb, dstb, rows, acc, sem_i0, sem_i1, sem_g, sem_s):
        c = lax.axis_index("c")
        s = lax.axis_index("s")
        rbase = s * RPT
        off = jnp.where(c == 0, s * nch, NS * nch)
        ngroups = jnp.where(c == 0, nch // ABUF, 0)

        def idx_start(j, sslot, dslot, sem):
            pltpu.async_copy(src_hbm.at[pl.ds(off + j * ABUF, ABUF)],
                             srcb.at[sslot], sem)
            pltpu.async_copy(dst_hbm.at[pl.ds(off + j * ABUF, ABUF)],
                             dstb.at[dslot], sem)

        def idx_wait(sem):
            for _ in range(2):
                pltpu.make_async_copy(src_hbm.at[pl.ds(0, ABUF)],
                                      srcb.at[0], sem).wait()

        def gather_start(p):
            for b in range(ABUF):
                pltpu.async_copy(table_hbm.at[srcb.at[p].at[b]],
                                 rows.at[p].at[b], sem_g)

        def gather_wait():
            for b in range(ABUF):
                pltpu.make_async_copy(table_hbm.at[pl.ds(0, CHUNK)],
                                      rows.at[0].at[b], sem_g).wait()

        def scatter_start(p, dslot):
            for b in range(ABUF):
                pltpu.async_copy(rows.at[p].at[b],
                                 acc.at[dstb.at[dslot].at[b]], sem_s, add=True)

        def scatter_wait():
            for b in range(ABUF):
                pltpu.make_async_copy(rows.at[0].at[b],
                                      acc.at[pl.ds(0, CHUNK)], sem_s).wait()

        zv = jnp.zeros((16,), jnp.float32)

        def zrow(i, carry):
            for kk in range(d // 16):
                rows[0, 0, i, pl.ds(kk * 16, 16)] = zv
            return carry

        lax.fori_loop(0, CHUNK, zrow, 0)
        for r5 in range(RPT // CHUNK):
            pltpu.sync_copy(rows.at[0].at[0],
                            acc.at[pl.ds(rbase + r5 * CHUNK, CHUNK)])

        plsc.subcore_barrier()

        idx_start(0, 0, 0, sem_i0)
        idx_start(1, 1, 1, sem_i1)
        idx_wait(sem_i0)
        gather_start(0)

        def quad(u, carry):
            for q in range(4):
                g = 4 * u + q
                p = q % 2
                sem_p = sem_i0 if p == 0 else sem_i1
                sem_o = sem_i1 if p == 0 else sem_i0
                gather_wait()
                if q == 0:
                    @pl.when(g > 0)
                    def _():
                        scatter_wait()
                else:
                    scatter_wait()

                @pl.when(g + 2 < ngroups)
                def _(sem_p=sem_p, g=g, p=p, q=q):
                    idx_start(g + 2, p, (q + 2) % 4, sem_p)

                scatter_start(p, q)

                @pl.when(g + 1 < ngroups)
                def _(sem_o=sem_o, p=p):
                    idx_wait(sem_o)
                    gather_start(1 - p)

            return carry

        lax.fori_loop(0, ngroups // 4, quad, 0)

        @pl.when(ngroups > 0)
        def _():
            scatter_wait()

        @pl.when(ngroups == 0)
        def _():
            gather_wait()
            idx_wait(sem_i1)

        plsc.subcore_barrier()
        obase = c * N_PAD + rbase
        for r5 in range(RPT // CHUNK):
            if r5 >= 2:
                pltpu.make_async_copy(rows.at[0].at[0],
                                      out_hbm.at[pl.ds(0, CHUNK)],
                                      sem_g).wait()
            pltpu.sync_copy(acc.at[pl.ds(rbase + r5 * CHUNK, CHUNK)],
                            rows.at[r5 % 2].at[0])
            pltpu.async_copy(rows.at[r5 % 2].at[0],
                             out_hbm.at[pl.ds(obase + r5 * CHUNK, CHUNK)],
                             sem_g)
        for _ in range(2):
            pltpu.make_async_copy(rows.at[0].at[0],
                                  out_hbm.at[pl.ds(0, CHUNK)],
                                  sem_g).wait()

    return k


_BM = 1024
_GRID = (N_PAD // _BM,)


def _tc_h_body(x_ref, w_ref, o_ref):
    o_ref[...] = jnp.dot(x_ref[...], w_ref[...],
                         preferred_element_type=jnp.float32)


def _tc_scale_body(h_ref, dp_ref, o_ref, dv_ref):
    deg = jnp.sum(dp_ref[...], axis=0, keepdims=True) + 1.0
    dinv = jnp.transpose(lax.rsqrt(deg))
    o_ref[...] = h_ref[...] * dinv
    dv_ref[...] = jnp.broadcast_to(dinv, (dinv.shape[0], 16))


def _tc_mid_body(a_ref, h2_ref, dv_ref, b1_ref, w_ref, o_ref):
    dinv = dv_ref[:, 0:1]
    a = (a_ref[...] + h2_ref[...]) * dinv + b1_ref[...]
    hr = jnp.maximum(a, 0.0)
    g = jnp.dot(hr, w_ref[...], preferred_element_type=jnp.float32)
    o_ref[...] = g * dinv


def _tc_out_body(a_ref, g2_ref, dv_ref, b2_ref, ls_ref, lg_ref):
    dinv = dv_ref[:, 0:1]
    logits = (a_ref[...] + g2_ref[...]) * dinv + b2_ref[...]
    col = lax.broadcasted_iota(jnp.int32, (_BM, C_PAD), 1)
    valid = col < N_CLASSES
    m = jnp.max(jnp.where(valid, logits, -1e30), axis=1, keepdims=True)
    e = jnp.where(valid, jnp.exp(logits - m), 0.0)
    ssum = jnp.sum(e, axis=1, keepdims=True)
    ls_ref[...] = logits - m - jnp.log(ssum)
    lg_ref[...] = logits


def kernel(x, edge_index, W1, b1, W2, b2):
    src = edge_index[0].astype(jnp.int32)
    dst = edge_index[1].astype(jnp.int32)
    n_edges = src.shape[0]
    per_s = -(-n_edges // (NS * CHUNK))
    per_s = -(-per_s // 16) * 16
    e_pad = (NS * per_s + 16) * CHUNK
    pad = jnp.full((e_pad - n_edges,), N_NODES, dtype=jnp.int32)
    src2 = jnp.concatenate([src, pad]).reshape(NS * per_s + 16, CHUNK)
    dst2 = jnp.concatenate([dst, pad]).reshape(NS * per_s + 16, CHUNK)

    xp = jnp.zeros((N_PAD, D_FEAT), jnp.float32).at[:N_NODES].set(x)
    zeros1 = jnp.zeros((N_PAD,), jnp.float32)
    W2p = jnp.zeros((HIDDEN, C_PAD), jnp.float32).at[:, :N_CLASSES].set(W2)
    b1r = b1.reshape(1, HIDDEN)
    b2r = jnp.zeros((1, C_PAD), jnp.float32).at[0, :N_CLASSES].set(b2)

    degp = _make_deg_kernel(per_s)(zeros1, dst2)

    h = pl.pallas_call(
        _tc_h_body,
        grid=_GRID,
        in_specs=[
            pl.BlockSpec((_BM, D_FEAT), lambda i: (i, 0)),
            pl.BlockSpec((D_FEAT, HIDDEN), lambda i: (0, 0)),
        ],
        out_specs=pl.BlockSpec((_BM, HIDDEN), lambda i: (i, 0)),
        out_shape=jax.ShapeDtypeStruct((N_PAD, HIDDEN), jnp.float32),
    )(xp, W1)

    h2, dinv16 = pl.pallas_call(
        _tc_scale_body,
        grid=_GRID,
        in_specs=[
            pl.BlockSpec((_BM, HIDDEN), lambda i: (i, 0)),
            pl.BlockSpec((NW, _BM), lambda i: (0, i)),
        ],
        out_specs=[
            pl.BlockSpec((_BM, HIDDEN), lambda i: (i, 0)),
            pl.BlockSpec((_BM, 16), lambda i: (i, 0)),
        ],
        out_shape=[
            jax.ShapeDtypeStruct((N_PAD, HIDDEN), jnp.float32),
            jax.ShapeDtypeStruct((N_PAD, 16), jnp.float32),
        ],
    )(h, degp)

    agg1 = _make_agg_kernel(HIDDEN, per_s)(h2, src2, dst2)[:N_PAD]

    g2 = pl.pallas_call(
        _tc_mid_body,
        grid=_GRID,
        in_specs=[
            pl.BlockSpec((_BM, HIDDEN), lambda i: (i, 0)),
            pl.BlockSpec((_BM, HIDDEN), lambda i: (i, 0)),
            pl.BlockSpec((_BM, 16), lambda i: (i, 0)),
            pl.BlockSpec((1, HIDDEN), lambda i: (0, 0)),
            pl.BlockSpec((HIDDEN, C_PAD), lambda i: (0, 0)),
        ],
        out_specs=pl.BlockSpec((_BM, C_PAD), lambda i: (i, 0)),
        out_shape=jax.ShapeDtypeStruct((N_PAD, C_PAD), jnp.float32),
    )(agg1, h2, dinv16, b1r, W2p)

    agg2 = _make_agg_kernel(C_PAD, per_s)(g2, src2, dst2)[:N_PAD]

    ls, lg = pl.pallas_call(
        _tc_out_body,
        grid=_GRID,
        in_specs=[
            pl.BlockSpec((_BM, C_PAD), lambda i: (i, 0)),
            pl.BlockSpec((_BM, C_PAD), lambda i: (i, 0)),
            pl.BlockSpec((_BM, 16), lambda i: (i, 0)),
            pl.BlockSpec((1, C_PAD), lambda i: (0, 0)),
        ],
        out_specs=[
            pl.BlockSpec((_BM, C_PAD), lambda i: (i, 0)),
            pl.BlockSpec((_BM, C_PAD), lambda i: (i, 0)),
        ],
        out_shape=[
            jax.ShapeDtypeStruct((N_PAD, C_PAD), jnp.float32),
            jax.ShapeDtypeStruct((N_PAD, C_PAD), jnp.float32),
        ],
    )(agg2, g2, dinv16, b2r)

    return (ls[:N_NODES, :N_CLASSES], lg[:N_NODES, :N_CLASSES])


# speedup vs baseline: 1.9479x; 1.9479x over previous
import functools

import jax
import jax.numpy as jnp
from jax import lax
from jax.experimental import pallas as pl
from jax.experimental.pallas import tpu as pltpu
from jax.experimental.pallas import tpu_sc as plsc

N_NODES = 10000
D_FEAT = 128
HIDDEN = 64
N_CLASSES = 47
C_PAD = 48

NC, NS = 2, 16
NW = NC * NS
CHUNK = 128
ABUF = 4
N_PAD = 10240
RPT = N_PAD // NS


def _sc_mesh():
    return plsc.VectorSubcoreMesh(core_axis_name="c", subcore_axis_name="s")


def _split(per_s, f):
    nch0 = min(per_s, max(0, int(round(f * per_s / 16)) * 16))
    return nch0, per_s - nch0


_SC_PARAMS = dict(
    compiler_params=pltpu.CompilerParams(use_tc_tiling_on_sc=False,
                                         needs_layout_passes=False),
)


def _make_deg_kernel(per_s):
    nch = per_s // 2

    @functools.partial(
        pl.kernel,
        out_type=jax.ShapeDtypeStruct((NW, N_PAD), jnp.float32),
        mesh=_sc_mesh(),
        scratch_types=[
            pltpu.VMEM((nch, CHUNK), jnp.int32),
            pltpu.VMEM((N_PAD,), jnp.float32),
        ],
        **_SC_PARAMS,
    )
    def k(zeros_hbm, dst_hbm, out_hbm, dst_v, hist):
        c = lax.axis_index("c")
        s = lax.axis_index("s")
        wid = s * NC + c
        pltpu.sync_copy(dst_hbm.at[pl.ds(wid * nch, nch)], dst_v)
        pltpu.sync_copy(zeros_hbm, hist)
        ones = jnp.ones((16,), jnp.float32)

        def chunk(j, carry):
            for kk in range(CHUNK // 16):
                idx = dst_v[j, pl.ds(kk * 16, 16)]
                plsc.addupdate_scatter(hist, [idx], ones)
            return carry

        lax.fori_loop(0, nch, chunk, 0)
        pltpu.sync_copy(hist, out_hbm.at[wid])

    return k


def _make_agg_kernel(d, nch0, nch1):
    assert nch0 % (4 * ABUF) == 0 and nch1 % (4 * ABUF) == 0

    @functools.partial(
        pl.kernel,
        out_type=jax.ShapeDtypeStruct((NC * N_PAD, d), jnp.float32),
        mesh=_sc_mesh(),
        scratch_types=[
            pltpu.VMEM((2, ABUF, CHUNK), jnp.int32),
            pltpu.VMEM((4, ABUF, CHUNK), jnp.int32),
            pltpu.VMEM((2, ABUF, CHUNK, d), jnp.float32),
            pltpu.VMEM_SHARED((N_PAD, d), jnp.float32),
            pltpu.SemaphoreType.DMA,
            pltpu.SemaphoreType.DMA,
            pltpu.SemaphoreType.DMA,
            pltpu.SemaphoreType.DMA,
        ],
        **_SC_PARAMS,
    )
    def k(table_hbm, src_hbm, dst_hbm, out_hbm,
          srcb, dstb, rows, acc, sem_i0, sem_i1, sem_g, sem_s):
        c = lax.axis_index("c")
        s = lax.axis_index("s")
        rbase = s * RPT
        off = jnp.where(c == 0, s * nch0, NS * nch0 + s * nch1)
        ngroups = jnp.where(c == 0, nch0 // ABUF, nch1 // ABUF)

        def idx_start(j, sslot, dslot, sem):
            pltpu.async_copy(src_hbm.at[pl.ds(off + j * ABUF, ABUF)],
                             srcb.at[sslot], sem)
            pltpu.async_copy(dst_hbm.at[pl.ds(off + j * ABUF, ABUF)],
                             dstb.at[dslot], sem)

        def idx_wait(sem):
            for _ in range(2):
                pltpu.make_async_copy(src_hbm.at[pl.ds(0, ABUF)],
                                      srcb.at[0], sem).wait()

        def gather_start(p):
            for b in range(ABUF):
                pltpu.async_copy(table_hbm.at[srcb.at[p].at[b]],
                                 rows.at[p].at[b], sem_g)

        def gather_wait():
            for b in range(ABUF):
                pltpu.make_async_copy(table_hbm.at[pl.ds(0, CHUNK)],
                                      rows.at[0].at[b], sem_g).wait()

        def scatter_start(p, dslot):
            for b in range(ABUF):
                pltpu.async_copy(rows.at[p].at[b],
                                 acc.at[dstb.at[dslot].at[b]], sem_s, add=True)

        def scatter_wait():
            for b in range(ABUF):
                pltpu.make_async_copy(rows.at[0].at[b],
                                      acc.at[pl.ds(0, CHUNK)], sem_s).wait()

        zv = jnp.zeros((16,), jnp.float32)

        def zrow(i, carry):
            for kk in range(d // 16):
                rows[0, 0, i, pl.ds(kk * 16, 16)] = zv
            return carry

        lax.fori_loop(0, CHUNK, zrow, 0)
        for r5 in range(RPT // CHUNK):
            pltpu.sync_copy(rows.at[0].at[0],
                            acc.at[pl.ds(rbase + r5 * CHUNK, CHUNK)])

        plsc.subcore_barrier()

        idx_start(0, 0, 0, sem_i0)
        idx_start(1, 1, 1, sem_i1)
        idx_wait(sem_i0)
        gather_start(0)

        def quad(u, carry):
            for q in range(4):
                g = 4 * u + q
                p = q % 2
                sem_p = sem_i0 if p == 0 else sem_i1
                sem_o = sem_i1 if p == 0 else sem_i0
                gather_wait()
                if q == 0:
                    @pl.when(g > 0)
                    def _():
                        scatter_wait()
                else:
                    scatter_wait()

                @pl.when(g + 2 < ngroups)
                def _(sem_p=sem_p, g=g, p=p, q=q):
                    idx_start(g + 2, p, (q + 2) % 4, sem_p)

                scatter_start(p, q)

                @pl.when(g + 1 < ngroups)
                def _(sem_o=sem_o, p=p):
                    idx_wait(sem_o)
                    gather_start(1 - p)

            return carry

        lax.fori_loop(0, ngroups // 4, quad, 0)

        @pl.when(ngroups > 0)
        def _():
            scatter_wait()

        @pl.when(ngroups == 0)
        def _():
            gather_wait()
            idx_wait(sem_i1)

        plsc.subcore_barrier()
        obase = c * N_PAD + rbase
        for r5 in range(RPT // CHUNK):
            if r5 >= 2:
                pltpu.make_async_copy(rows.at[0].at[0],
                                      out_hbm.at[pl.ds(0, CHUNK)],
                                      sem_g).wait()
            pltpu.sync_copy(acc.at[pl.ds(rbase + r5 * CHUNK, CHUNK)],
                            rows.at[r5 % 2].at[0])
            pltpu.async_copy(rows.at[r5 % 2].at[0],
                             out_hbm.at[pl.ds(obase + r5 * CHUNK, CHUNK)],
                             sem_g)
        for _ in range(2):
            pltpu.make_async_copy(rows.at[0].at[0],
                                  out_hbm.at[pl.ds(0, CHUNK)],
                                  sem_g).wait()

    return k


_BM = 1024
_GRID = (N_PAD // _BM,)


def _tc_h_body(x_ref, w_ref, o_ref):
    o_ref[...] = jnp.dot(x_ref[...], w_ref[...],
                         preferred_element_type=jnp.float32)


def _tc_scale_body(h_ref, dp_ref, o_ref, dv_ref):
    deg = jnp.sum(dp_ref[...], axis=0, keepdims=True) + 1.0
    dinv = jnp.transpose(lax.rsqrt(deg))
    o_ref[...] = h_ref[...] * dinv
    dv_ref[...] = jnp.broadcast_to(dinv, (dinv.shape[0], 16))


def _tc_mid_body(a_ref, h2_ref, dv_ref, b1_ref, w_ref, o_ref):
    dinv = dv_ref[:, 0:1]
    a = (a_ref[0] + a_ref[1] + h2_ref[...]) * dinv + b1_ref[...]
    hr = jnp.maximum(a, 0.0)
    g = jnp.dot(hr, w_ref[...], preferred_element_type=jnp.float32)
    o_ref[...] = g * dinv


def _tc_out_body(a_ref, g2_ref, dv_ref, b2_ref, ls_ref, lg_ref):
    dinv = dv_ref[:, 0:1]
    logits = (a_ref[0] + a_ref[1] + g2_ref[...]) * dinv + b2_ref[...]
    col = lax.broadcasted_iota(jnp.int32, (_BM, C_PAD), 1)
    valid = col < N_CLASSES
    m = jnp.max(jnp.where(valid, logits, -1e30), axis=1, keepdims=True)
    e = jnp.where(valid, jnp.exp(logits - m), 0.0)
    ssum = jnp.sum(e, axis=1, keepdims=True)
    ls_ref[...] = logits - m - jnp.log(ssum)
    lg_ref[...] = logits


def kernel(x, edge_index, W1, b1, W2, b2):
    src = edge_index[0].astype(jnp.int32)
    dst = edge_index[1].astype(jnp.int32)
    n_edges = src.shape[0]
    per_s = -(-n_edges // (NS * CHUNK))
    per_s = -(-per_s // 16) * 16
    e_pad = (NS * per_s + 16) * CHUNK
    pad = jnp.full((e_pad - n_edges,), N_NODES, dtype=jnp.int32)
    src2 = jnp.concatenate([src, pad]).reshape(NS * per_s + 16, CHUNK)
    dst2 = jnp.concatenate([dst, pad]).reshape(NS * per_s + 16, CHUNK)

    xp = jnp.zeros((N_PAD, D_FEAT), jnp.float32).at[:N_NODES].set(x)
    zeros1 = jnp.zeros((N_PAD,), jnp.float32)
    W2p = jnp.zeros((HIDDEN, C_PAD), jnp.float32).at[:, :N_CLASSES].set(W2)
    b1r = b1.reshape(1, HIDDEN)
    b2r = jnp.zeros((1, C_PAD), jnp.float32).at[0, :N_CLASSES].set(b2)

    degp = _make_deg_kernel(per_s)(zeros1, dst2)

    h = pl.pallas_call(
        _tc_h_body,
        grid=_GRID,
        in_specs=[
            pl.BlockSpec((_BM, D_FEAT), lambda i: (i, 0)),
            pl.BlockSpec((D_FEAT, HIDDEN), lambda i: (0, 0)),
        ],
        out_specs=pl.BlockSpec((_BM, HIDDEN), lambda i: (i, 0)),
        out_shape=jax.ShapeDtypeStruct((N_PAD, HIDDEN), jnp.float32),
    )(xp, W1)

    h2, dinv16 = pl.pallas_call(
        _tc_scale_body,
        grid=_GRID,
        in_specs=[
            pl.BlockSpec((_BM, HIDDEN), lambda i: (i, 0)),
            pl.BlockSpec((NW, _BM), lambda i: (0, i)),
        ],
        out_specs=[
            pl.BlockSpec((_BM, HIDDEN), lambda i: (i, 0)),
            pl.BlockSpec((_BM, 16), lambda i: (i, 0)),
        ],
        out_shape=[
            jax.ShapeDtypeStruct((N_PAD, HIDDEN), jnp.float32),
            jax.ShapeDtypeStruct((N_PAD, 16), jnp.float32),
        ],
    )(h, degp)

    a0, a1 = _split(per_s, 0.9)
    agg1 = _make_agg_kernel(HIDDEN, a0, a1)(h2, src2, dst2)
    agg1 = agg1.reshape(NC, N_PAD, HIDDEN)

    g2 = pl.pallas_call(
        _tc_mid_body,
        grid=_GRID,
        in_specs=[
            pl.BlockSpec((NC, _BM, HIDDEN), lambda i: (0, i, 0)),
            pl.BlockSpec((_BM, HIDDEN), lambda i: (i, 0)),
            pl.BlockSpec((_BM, 16), lambda i: (i, 0)),
            pl.BlockSpec((1, HIDDEN), lambda i: (0, 0)),
            pl.BlockSpec((HIDDEN, C_PAD), lambda i: (0, 0)),
        ],
        out_specs=pl.BlockSpec((_BM, C_PAD), lambda i: (i, 0)),
        out_shape=jax.ShapeDtypeStruct((N_PAD, C_PAD), jnp.float32),
    )(agg1, h2, dinv16, b1r, W2p)

    agg2 = _make_agg_kernel(C_PAD, a0, a1)(g2, src2, dst2)
    agg2 = agg2.reshape(NC, N_PAD, C_PAD)

    ls, lg = pl.pallas_call(
        _tc_out_body,
        grid=_GRID,
        in_specs=[
            pl.BlockSpec((NC, _BM, C_PAD), lambda i: (0, i, 0)),
            pl.BlockSpec((_BM, C_PAD), lambda i: (i, 0)),
            pl.BlockSpec((_BM, 16), lambda i: (i, 0)),
            pl.BlockSpec((1, C_PAD), lambda i: (0, 0)),
        ],
        out_specs=[
            pl.BlockSpec((_BM, C_PAD), lambda i: (i, 0)),
            pl.BlockSpec((_BM, C_PAD), lambda i: (i, 0)),
        ],
        out_shape=[
            jax.ShapeDtypeStruct((N_PAD, C_PAD), jnp.float32),
            jax.ShapeDtypeStruct((N_PAD, C_PAD), jnp.float32),
        ],
    )(agg2, g2, dinv16, b2r)

    return (ls[:N_NODES, :N_CLASSES], lg[:N_NODES, :N_CLASSES])


# speedup vs baseline: 3.0713x; 1.5768x over previous
import functools

import jax
import jax.numpy as jnp
from jax import lax
from jax.experimental import pallas as pl
from jax.experimental.pallas import tpu as pltpu
from jax.experimental.pallas import tpu_sc as plsc

N_NODES = 10000
D_FEAT = 128
HIDDEN = 64
N_CLASSES = 47
C_PAD = 48

NC, NS = 2, 16
NW = NC * NS
CHUNK = 128
ABUF = 4
N_PAD = 10240
RPT = N_PAD // NS


def _sc_mesh():
    return plsc.VectorSubcoreMesh(core_axis_name="c", subcore_axis_name="s")


def _split(per_s, f):
    nch0 = min(per_s, max(0, int(round(f * per_s / 16)) * 16))
    return nch0, per_s - nch0


_SC_PARAMS = dict(
    compiler_params=pltpu.CompilerParams(use_tc_tiling_on_sc=False,
                                         needs_layout_passes=False),
)


def _make_deg_kernel(per_s):
    nch = per_s // 2

    @functools.partial(
        pl.kernel,
        out_type=jax.ShapeDtypeStruct((NW, N_PAD), jnp.float32),
        mesh=_sc_mesh(),
        scratch_types=[
            pltpu.VMEM((nch, CHUNK), jnp.int32),
            pltpu.VMEM((N_PAD,), jnp.float32),
        ],
        **_SC_PARAMS,
    )
    def k(zeros_hbm, dst_hbm, out_hbm, dst_v, hist):
        c = lax.axis_index("c")
        s = lax.axis_index("s")
        wid = s * NC + c
        pltpu.sync_copy(dst_hbm.at[pl.ds(wid * nch, nch)], dst_v)
        pltpu.sync_copy(zeros_hbm, hist)
        ones = jnp.ones((16,), jnp.float32)

        def chunk(j, carry):
            for kk in range(CHUNK // 16):
                idx = dst_v[j, pl.ds(kk * 16, 16)]
                plsc.addupdate_scatter(hist, [idx], ones)
            return carry

        lax.fori_loop(0, nch, chunk, 0)
        pltpu.sync_copy(hist, out_hbm.at[wid])

    return k


def _make_agg_kernel(d, nch0, nch1):
    assert nch0 % (4 * ABUF) == 0 and nch1 % (4 * ABUF) == 0

    @functools.partial(
        pl.kernel,
        out_type=jax.ShapeDtypeStruct((NC * N_PAD, d), jnp.float32),
        mesh=_sc_mesh(),
        scratch_types=[
            pltpu.VMEM((2, ABUF, CHUNK), jnp.int32),
            pltpu.VMEM((4, ABUF, CHUNK), jnp.int32),
            pltpu.VMEM((2, ABUF, CHUNK, d), jnp.float32),
            pltpu.VMEM_SHARED((N_PAD, d), jnp.float32),
            pltpu.SemaphoreType.DMA,
            pltpu.SemaphoreType.DMA,
            pltpu.SemaphoreType.DMA,
            pltpu.SemaphoreType.DMA,
        ],
        **_SC_PARAMS,
    )
    def k(table_hbm, src_hbm, dst_hbm, out_hbm,
          srcb, dstb, rows, acc, sem_i0, sem_i1, sem_g, sem_s):
        c = lax.axis_index("c")
        s = lax.axis_index("s")
        rbase = s * RPT
        off = jnp.where(c == 0, s * nch0, NS * nch0 + s * nch1)
        ngroups = jnp.where(c == 0, nch0 // ABUF, nch1 // ABUF)

        def idx_start(j, sslot, dslot, sem):
            pltpu.async_copy(src_hbm.at[pl.ds(off + j * ABUF, ABUF)],
                             srcb.at[sslot], sem)
            pltpu.async_copy(dst_hbm.at[pl.ds(off + j * ABUF, ABUF)],
                             dstb.at[dslot], sem)

        def idx_wait(sem):
            for _ in range(2):
                pltpu.make_async_copy(src_hbm.at[pl.ds(0, ABUF)],
                                      srcb.at[0], sem).wait()

        def gather_start(p):
            for b in range(ABUF):
                pltpu.async_copy(table_hbm.at[srcb.at[p].at[b]],
                                 rows.at[p].at[b], sem_g)

        def gather_wait():
            for b in range(ABUF):
                pltpu.make_async_copy(table_hbm.at[pl.ds(0, CHUNK)],
                                      rows.at[0].at[b], sem_g).wait()

        def scatter_start(p, dslot):
            for b in range(ABUF):
                pltpu.async_copy(rows.at[p].at[b],
                                 acc.at[dstb.at[dslot].at[b]], sem_s, add=True)

        def scatter_wait():
            for b in range(ABUF):
                pltpu.make_async_copy(rows.at[0].at[b],
                                      acc.at[pl.ds(0, CHUNK)], sem_s).wait()

        zv = jnp.zeros((16,), jnp.float32)

        def zrow(i, carry):
            for kk in range(d // 16):
                rows[0, 0, i, pl.ds(kk * 16, 16)] = zv
            return carry

        lax.fori_loop(0, CHUNK, zrow, 0)
        for r5 in range(RPT // CHUNK):
            pltpu.sync_copy(rows.at[0].at[0],
                            acc.at[pl.ds(rbase + r5 * CHUNK, CHUNK)])

        plsc.subcore_barrier()

        idx_start(0, 0, 0, sem_i0)
        idx_start(1, 1, 1, sem_i1)
        idx_wait(sem_i0)
        gather_start(0)

        def quad(u, carry):
            for q in range(4):
                g = 4 * u + q
                p = q % 2
                sem_p = sem_i0 if p == 0 else sem_i1
                sem_o = sem_i1 if p == 0 else sem_i0
                gather_wait()
                if q == 0:
                    @pl.when(g > 0)
                    def _():
                        scatter_wait()
                else:
                    scatter_wait()

                @pl.when(g + 2 < ngroups)
                def _(sem_p=sem_p, g=g, p=p, q=q):
                    idx_start(g + 2, p, (q + 2) % 4, sem_p)

                scatter_start(p, q)

                @pl.when(g + 1 < ngroups)
                def _(sem_o=sem_o, p=p):
                    idx_wait(sem_o)
                    gather_start(1 - p)

            return carry

        lax.fori_loop(0, ngroups // 4, quad, 0)

        @pl.when(ngroups > 0)
        def _():
            scatter_wait()

        @pl.when(ngroups == 0)
        def _():
            gather_wait()
            idx_wait(sem_i1)

        plsc.subcore_barrier()
        obase = c * N_PAD + rbase
        for r5 in range(RPT // CHUNK):
            if r5 >= 2:
                pltpu.make_async_copy(rows.at[0].at[0],
                                      out_hbm.at[pl.ds(0, CHUNK)],
                                      sem_g).wait()
            pltpu.sync_copy(acc.at[pl.ds(rbase + r5 * CHUNK, CHUNK)],
                            rows.at[r5 % 2].at[0])
            pltpu.async_copy(rows.at[r5 % 2].at[0],
                             out_hbm.at[pl.ds(obase + r5 * CHUNK, CHUNK)],
                             sem_g)
        for _ in range(2):
            pltpu.make_async_copy(rows.at[0].at[0],
                                  out_hbm.at[pl.ds(0, CHUNK)],
                                  sem_g).wait()

    return k


_BM = 1024
_GRID = (N_PAD // _BM,)


def _tc_h_body(x_ref, w_ref, o_ref):
    o_ref[...] = jnp.dot(x_ref[...], w_ref[...],
                         preferred_element_type=jnp.float32)


def _tc_scale_body(h_ref, dp_ref, o_ref, dv_ref):
    deg = jnp.sum(dp_ref[...], axis=0, keepdims=True) + 1.0
    dinv = jnp.transpose(lax.rsqrt(deg))
    o_ref[...] = h_ref[...] * dinv
    dv_ref[...] = jnp.broadcast_to(dinv, (dinv.shape[0], 16))


def _tc_mid_body(a_ref, h2_ref, dv_ref, b1_ref, w_ref, o_ref):
    dinv = dv_ref[:, 0:1]
    a = (a_ref[0] + a_ref[1] + h2_ref[...]) * dinv + b1_ref[...]
    hr = jnp.maximum(a, 0.0)
    g = jnp.dot(hr, w_ref[...], preferred_element_type=jnp.float32)
    o_ref[...] = g * dinv


def _tc_out_body(a_ref, g2_ref, dv_ref, b2_ref, ls_ref, lg_ref):
    dinv = dv_ref[:, 0:1]
    logits = (a_ref[0] + a_ref[1] + g2_ref[...]) * dinv + b2_ref[...]
    col = lax.broadcasted_iota(jnp.int32, (_BM, C_PAD), 1)
    valid = col < N_CLASSES
    m = jnp.max(jnp.where(valid, logits, -1e30), axis=1, keepdims=True)
    e = jnp.where(valid, jnp.exp(logits - m), 0.0)
    ssum = jnp.sum(e, axis=1, keepdims=True)
    ls_ref[...] = logits - m - jnp.log(ssum)
    lg_ref[...] = logits


def kernel(x, edge_index, W1, b1, W2, b2):
    src = edge_index[0].astype(jnp.int32)
    dst = edge_index[1].astype(jnp.int32)
    n_edges = src.shape[0]
    per_s = -(-n_edges // (NS * CHUNK))
    per_s = -(-per_s // 16) * 16
    e_pad = NS * per_s * CHUNK
    npad_e = e_pad - n_edges
    ar = jnp.arange(npad_e, dtype=jnp.int32)
    pad_src = ar % N_NODES
    pad_dst = N_NODES + ar % (N_PAD - N_NODES)
    src2 = jnp.concatenate([src, pad_src]).reshape(NS * per_s, CHUNK)
    dst2 = jnp.concatenate([dst, pad_dst]).reshape(NS * per_s, CHUNK)

    xp = jnp.zeros((N_PAD, D_FEAT), jnp.float32).at[:N_NODES].set(x)
    zeros1 = jnp.zeros((N_PAD,), jnp.float32)
    W2p = jnp.zeros((HIDDEN, C_PAD), jnp.float32).at[:, :N_CLASSES].set(W2)
    b1r = b1.reshape(1, HIDDEN)
    b2r = jnp.zeros((1, C_PAD), jnp.float32).at[0, :N_CLASSES].set(b2)

    degp = _make_deg_kernel(per_s)(zeros1, dst2)

    h = pl.pallas_call(
        _tc_h_body,
        grid=_GRID,
        in_specs=[
            pl.BlockSpec((_BM, D_FEAT), lambda i: (i, 0)),
            pl.BlockSpec((D_FEAT, HIDDEN), lambda i: (0, 0)),
        ],
        out_specs=pl.BlockSpec((_BM, HIDDEN), lambda i: (i, 0)),
        out_shape=jax.ShapeDtypeStruct((N_PAD, HIDDEN), jnp.float32),
    )(xp, W1)

    h2, dinv16 = pl.pallas_call(
        _tc_scale_body,
        grid=_GRID,
        in_specs=[
            pl.BlockSpec((_BM, HIDDEN), lambda i: (i, 0)),
            pl.BlockSpec((NW, _BM), lambda i: (0, i)),
        ],
        out_specs=[
            pl.BlockSpec((_BM, HIDDEN), lambda i: (i, 0)),
            pl.BlockSpec((_BM, 16), lambda i: (i, 0)),
        ],
        out_shape=[
            jax.ShapeDtypeStruct((N_PAD, HIDDEN), jnp.float32),
            jax.ShapeDtypeStruct((N_PAD, 16), jnp.float32),
        ],
    )(h, degp)

    a0, a1 = _split(per_s, 0.9)
    agg1 = _make_agg_kernel(HIDDEN, a0, a1)(h2, src2, dst2)
    agg1 = agg1.reshape(NC, N_PAD, HIDDEN)

    g2 = pl.pallas_call(
        _tc_mid_body,
        grid=_GRID,
        in_specs=[
            pl.BlockSpec((NC, _BM, HIDDEN), lambda i: (0, i, 0)),
            pl.BlockSpec((_BM, HIDDEN), lambda i: (i, 0)),
            pl.BlockSpec((_BM, 16), lambda i: (i, 0)),
            pl.BlockSpec((1, HIDDEN), lambda i: (0, 0)),
            pl.BlockSpec((HIDDEN, C_PAD), lambda i: (0, 0)),
        ],
        out_specs=pl.BlockSpec((_BM, C_PAD), lambda i: (i, 0)),
        out_shape=jax.ShapeDtypeStruct((N_PAD, C_PAD), jnp.float32),
    )(agg1, h2, dinv16, b1r, W2p)

    agg2 = _make_agg_kernel(C_PAD, a0, a1)(g2, src2, dst2)
    agg2 = agg2.reshape(NC, N_PAD, C_PAD)

    ls, lg = pl.pallas_call(
        _tc_out_body,
        grid=_GRID,
        in_specs=[
            pl.BlockSpec((NC, _BM, C_PAD), lambda i: (0, i, 0)),
            pl.BlockSpec((_BM, C_PAD), lambda i: (i, 0)),
            pl.BlockSpec((_BM, 16), lambda i: (i, 0)),
            pl.BlockSpec((1, C_PAD), lambda i: (0, 0)),
        ],
        out_specs=[
            pl.BlockSpec((_BM, C_PAD), lambda i: (i, 0)),
            pl.BlockSpec((_BM, C_PAD), lambda i: (i, 0)),
        ],
        out_shape=[
            jax.ShapeDtypeStruct((N_PAD, C_PAD), jnp.float32),
            jax.ShapeDtypeStruct((N_PAD, C_PAD), jnp.float32),
        ],
    )(agg2, g2, dinv16, b2r)

    return (ls[:N_NODES, :N_CLASSES], lg[:N_NODES, :N_CLASSES])


# speedup vs baseline: 3.6561x; 1.1904x over previous
import functools

import jax
import jax.numpy as jnp
from jax import lax
from jax.experimental import pallas as pl
from jax.experimental.pallas import tpu as pltpu
from jax.experimental.pallas import tpu_sc as plsc

N_NODES = 10000
D_FEAT = 128
HIDDEN = 64
N_CLASSES = 47
C_PAD = 48

NC, NS = 2, 16
NW = NC * NS
CHUNK = 128
ABUF = 4
N_PAD = 10240
RPT = N_PAD // NS


def _sc_mesh():
    return plsc.VectorSubcoreMesh(core_axis_name="c", subcore_axis_name="s")


def _split(per_s, f):
    nch0 = min(per_s, max(0, int(round(f * per_s / 16)) * 16))
    return nch0, per_s - nch0


_SC_PARAMS = dict(
    compiler_params=pltpu.CompilerParams(use_tc_tiling_on_sc=False,
                                         needs_layout_passes=False),
)


def _make_deg_kernel(per_s):
    nch = per_s // 2

    @functools.partial(
        pl.kernel,
        out_type=jax.ShapeDtypeStruct((NW, N_PAD), jnp.float32),
        mesh=_sc_mesh(),
        scratch_types=[
            pltpu.VMEM((nch, CHUNK), jnp.int32),
            pltpu.VMEM((N_PAD,), jnp.float32),
        ],
        **_SC_PARAMS,
    )
    def k(zeros_hbm, dst_hbm, out_hbm, dst_v, hist):
        c = lax.axis_index("c")
        s = lax.axis_index("s")
        wid = s * NC + c
        pltpu.sync_copy(dst_hbm.at[pl.ds(wid * nch, nch)], dst_v)
        pltpu.sync_copy(zeros_hbm, hist)
        ones = jnp.ones((16,), jnp.float32)

        def chunk(j, carry):
            for kk in range(CHUNK // 16):
                idx = dst_v[j, pl.ds(kk * 16, 16)]
                plsc.addupdate_scatter(hist, [idx], ones)
            return carry

        lax.fori_loop(0, nch, chunk, 0)
        pltpu.sync_copy(hist, out_hbm.at[wid])

    return k


def _make_agg_kernel(d, nch0, nch1):
    assert nch0 % (4 * ABUF) == 0 and nch1 % (4 * ABUF) == 0

    @functools.partial(
        pl.kernel,
        out_type=jax.ShapeDtypeStruct((NC * N_PAD, d), jnp.float32),
        mesh=_sc_mesh(),
        scratch_types=[
            pltpu.VMEM((2, ABUF, CHUNK), jnp.int32),
            pltpu.VMEM((4, ABUF, CHUNK), jnp.int32),
            pltpu.VMEM((2, ABUF, CHUNK, d), jnp.float32),
            pltpu.VMEM_SHARED((N_PAD, d), jnp.float32),
            pltpu.SemaphoreType.DMA,
            pltpu.SemaphoreType.DMA,
            pltpu.SemaphoreType.DMA,
            pltpu.SemaphoreType.DMA,
        ],
        **_SC_PARAMS,
    )
    def k(table_hbm, src_hbm, dst_hbm, out_hbm,
          srcb, dstb, rows, acc, sem_i0, sem_i1, sem_g, sem_s):
        c = lax.axis_index("c")
        s = lax.axis_index("s")
        rbase = s * RPT
        off = jnp.where(c == 0, s * nch0, NS * nch0 + s * nch1)
        ngroups = jnp.where(c == 0, nch0 // ABUF, nch1 // ABUF)

        def idx_start(j, sslot, dslot, sem):
            pltpu.async_copy(src_hbm.at[pl.ds(off + j * ABUF, ABUF)],
                             srcb.at[sslot], sem)
            pltpu.async_copy(dst_hbm.at[pl.ds(off + j * ABUF, ABUF)],
                             dstb.at[dslot], sem)

        def idx_wait(sem):
            for _ in range(2):
                pltpu.make_async_copy(src_hbm.at[pl.ds(0, ABUF)],
                                      srcb.at[0], sem).wait()

        def gather_start(p):
            for b in range(ABUF):
                pltpu.async_copy(table_hbm.at[srcb.at[p].at[b]],
                                 rows.at[p].at[b], sem_g)

        def gather_wait():
            for b in range(ABUF):
                pltpu.make_async_copy(table_hbm.at[pl.ds(0, CHUNK)],
                                      rows.at[0].at[b], sem_g).wait()

        def scatter_start(p, dslot):
            for b in range(ABUF):
                pltpu.async_copy(rows.at[p].at[b],
                                 acc.at[dstb.at[dslot].at[b]], sem_s, add=True)

        def scatter_wait():
            for b in range(ABUF):
                pltpu.make_async_copy(rows.at[0].at[b],
                                      acc.at[pl.ds(0, CHUNK)], sem_s).wait()

        zv = jnp.zeros((16,), jnp.float32)

        def zrow(i, carry):
            for kk in range(d // 16):
                rows[0, 0, i, pl.ds(kk * 16, 16)] = zv
            return carry

        lax.fori_loop(0, CHUNK, zrow, 0)
        for r5 in range(RPT // CHUNK):
            pltpu.sync_copy(rows.at[0].at[0],
                            acc.at[pl.ds(rbase + r5 * CHUNK, CHUNK)])

        plsc.subcore_barrier()

        idx_start(0, 0, 0, sem_i0)
        idx_start(1, 1, 1, sem_i1)
        idx_wait(sem_i0)
        gather_start(0)

        def quad(u, carry):
            for q in range(4):
                g = 4 * u + q
                p = q % 2
                sem_p = sem_i0 if p == 0 else sem_i1
                sem_o = sem_i1 if p == 0 else sem_i0
                gather_wait()
                if q == 0:
                    @pl.when(g > 0)
                    def _():
                        scatter_wait()
                else:
                    scatter_wait()

                @pl.when(g + 2 < ngroups)
                def _(sem_p=sem_p, g=g, p=p, q=q):
                    idx_start(g + 2, p, (q + 2) % 4, sem_p)

                scatter_start(p, q)

                @pl.when(g + 1 < ngroups)
                def _(sem_o=sem_o, p=p):
                    idx_wait(sem_o)
                    gather_start(1 - p)

            return carry

        lax.fori_loop(0, ngroups // 4, quad, 0)

        @pl.when(ngroups > 0)
        def _():
            scatter_wait()

        @pl.when(ngroups == 0)
        def _():
            gather_wait()
            idx_wait(sem_i1)

        plsc.subcore_barrier()
        obase = c * N_PAD + rbase
        for r5 in range(RPT // CHUNK):
            if r5 >= 2:
                pltpu.make_async_copy(rows.at[0].at[0],
                                      out_hbm.at[pl.ds(0, CHUNK)],
                                      sem_g).wait()
            pltpu.sync_copy(acc.at[pl.ds(rbase + r5 * CHUNK, CHUNK)],
                            rows.at[r5 % 2].at[0])
            pltpu.async_copy(rows.at[r5 % 2].at[0],
                             out_hbm.at[pl.ds(obase + r5 * CHUNK, CHUNK)],
                             sem_g)
        for _ in range(2):
            pltpu.make_async_copy(rows.at[0].at[0],
                                  out_hbm.at[pl.ds(0, CHUNK)],
                                  sem_g).wait()

    return k


_BM = 1024
_GRID = (N_PAD // _BM,)


def _tc_h_body(x_ref, w_ref, o_ref):
    o_ref[...] = jnp.dot(x_ref[...], w_ref[...],
                         preferred_element_type=jnp.float32)


def _tc_scale_body(h_ref, dp_ref, o_ref, dv_ref):
    deg = jnp.sum(dp_ref[...], axis=0, keepdims=True) + 1.0
    dinv = jnp.transpose(lax.rsqrt(deg))
    o_ref[...] = h_ref[...] * dinv
    dv_ref[...] = jnp.broadcast_to(dinv, (dinv.shape[0], 16))


def _tc_mid_body(a0_ref, a1_ref, h2_ref, dv_ref, b1_ref, w_ref, o_ref):
    dinv = dv_ref[:, 0:1]
    a = (a0_ref[...] + a1_ref[...] + h2_ref[...]) * dinv + b1_ref[...]
    hr = jnp.maximum(a, 0.0)
    g = jnp.dot(hr, w_ref[...], preferred_element_type=jnp.float32)
    o_ref[...] = g * dinv


def _tc_out_body(a0_ref, a1_ref, g2_ref, dv_ref, b2_ref, ls_ref, lg_ref):
    dinv = dv_ref[:, 0:1]
    logits = (a0_ref[...] + a1_ref[...] + g2_ref[...]) * dinv + b2_ref[...]
    col = lax.broadcasted_iota(jnp.int32, (_BM, C_PAD), 1)
    valid = col < N_CLASSES
    m = jnp.max(jnp.where(valid, logits, -1e30), axis=1, keepdims=True)
    e = jnp.where(valid, jnp.exp(logits - m), 0.0)
    ssum = jnp.sum(e, axis=1, keepdims=True)
    ls_ref[...] = logits - m - jnp.log(ssum)
    lg_ref[...] = logits


def kernel(x, edge_index, W1, b1, W2, b2):
    src = edge_index[0].astype(jnp.int32)
    dst = edge_index[1].astype(jnp.int32)
    n_edges = src.shape[0]
    per_s = -(-n_edges // (NS * CHUNK))
    per_s = -(-per_s // 16) * 16
    e_pad = NS * per_s * CHUNK
    npad_e = e_pad - n_edges
    ar = jnp.arange(npad_e, dtype=jnp.int32)
    pad_src = ar % N_NODES
    pad_dst = N_NODES + ar % (N_PAD - N_NODES)
    src2 = jnp.concatenate([src, pad_src]).reshape(NS * per_s, CHUNK)
    dst2 = jnp.concatenate([dst, pad_dst]).reshape(NS * per_s, CHUNK)

    xp = jnp.zeros((N_PAD, D_FEAT), jnp.float32).at[:N_NODES].set(x)
    zeros1 = jnp.zeros((N_PAD,), jnp.float32)
    W2p = jnp.zeros((HIDDEN, C_PAD), jnp.float32).at[:, :N_CLASSES].set(W2)
    b1r = b1.reshape(1, HIDDEN)
    b2r = jnp.zeros((1, C_PAD), jnp.float32).at[0, :N_CLASSES].set(b2)

    degp = _make_deg_kernel(per_s)(zeros1, dst2)

    h = pl.pallas_call(
        _tc_h_body,
        grid=_GRID,
        in_specs=[
            pl.BlockSpec((_BM, D_FEAT), lambda i: (i, 0)),
            pl.BlockSpec((D_FEAT, HIDDEN), lambda i: (0, 0)),
        ],
        out_specs=pl.BlockSpec((_BM, HIDDEN), lambda i: (i, 0)),
        out_shape=jax.ShapeDtypeStruct((N_PAD, HIDDEN), jnp.float32),
    )(xp, W1)

    h2, dinv16 = pl.pallas_call(
        _tc_scale_body,
        grid=_GRID,
        in_specs=[
            pl.BlockSpec((_BM, HIDDEN), lambda i: (i, 0)),
            pl.BlockSpec((NW, _BM), lambda i: (0, i)),
        ],
        out_specs=[
            pl.BlockSpec((_BM, HIDDEN), lambda i: (i, 0)),
            pl.BlockSpec((_BM, 16), lambda i: (i, 0)),
        ],
        out_shape=[
            jax.ShapeDtypeStruct((N_PAD, HIDDEN), jnp.float32),
            jax.ShapeDtypeStruct((N_PAD, 16), jnp.float32),
        ],
    )(h, degp)

    a0, a1 = _split(per_s, 0.6)
    agg1 = _make_agg_kernel(HIDDEN, a0, a1)(h2, src2, dst2)

    g2 = pl.pallas_call(
        _tc_mid_body,
        grid=_GRID,
        in_specs=[
            pl.BlockSpec((_BM, HIDDEN), lambda i: (i, 0)),
            pl.BlockSpec((_BM, HIDDEN), lambda i: (N_PAD // _BM + i, 0)),
            pl.BlockSpec((_BM, HIDDEN), lambda i: (i, 0)),
            pl.BlockSpec((_BM, 16), lambda i: (i, 0)),
            pl.BlockSpec((1, HIDDEN), lambda i: (0, 0)),
            pl.BlockSpec((HIDDEN, C_PAD), lambda i: (0, 0)),
        ],
        out_specs=pl.BlockSpec((_BM, C_PAD), lambda i: (i, 0)),
        out_shape=jax.ShapeDtypeStruct((N_PAD, C_PAD), jnp.float32),
    )(agg1, agg1, h2, dinv16, b1r, W2p)

    agg2 = _make_agg_kernel(C_PAD, a0, a1)(g2, src2, dst2)

    ls, lg = pl.pallas_call(
        _tc_out_body,
        grid=_GRID,
        in_specs=[
            pl.BlockSpec((_BM, C_PAD), lambda i: (i, 0)),
            pl.BlockSpec((_BM, C_PAD), lambda i: (N_PAD // _BM + i, 0)),
            pl.BlockSpec((_BM, C_PAD), lambda i: (i, 0)),
            pl.BlockSpec((_BM, 16), lambda i: (i, 0)),
            pl.BlockSpec((1, C_PAD), lambda i: (0, 0)),
        ],
        out_specs=[
            pl.BlockSpec((_BM, C_PAD), lambda i: (i, 0)),
            pl.BlockSpec((_BM, C_PAD), lambda i: (i, 0)),
        ],
        out_shape=[
            jax.ShapeDtypeStruct((N_PAD, C_PAD), jnp.float32),
            jax.ShapeDtypeStruct((N_PAD, C_PAD), jnp.float32),
        ],
    )(agg2, agg2, g2, dinv16, b2r)

    return (ls[:N_NODES, :N_CLASSES], lg[:N_NODES, :N_CLASSES])


# speedup vs baseline: 3.9178x; 1.0716x over previous
import functools

import jax
import jax.numpy as jnp
from jax import lax
from jax.experimental import pallas as pl
from jax.experimental.pallas import tpu as pltpu
from jax.experimental.pallas import tpu_sc as plsc

N_NODES = 10000
D_FEAT = 128
HIDDEN = 64
N_CLASSES = 47
C_PAD = 48

NC, NS = 2, 16
NW = NC * NS
CHUNK = 128
ABUF = 4
N_PAD = 10240
RPT = N_PAD // NS


def _sc_mesh():
    return plsc.VectorSubcoreMesh(core_axis_name="c", subcore_axis_name="s")


def _split(per_s, f):
    nch0 = min(per_s, max(0, int(round(f * per_s / 16)) * 16))
    return nch0, per_s - nch0


_SC_PARAMS = dict(
    compiler_params=pltpu.CompilerParams(use_tc_tiling_on_sc=False,
                                         needs_layout_passes=False),
)


def _make_deg_kernel(per_s):
    nch = per_s // 2

    @functools.partial(
        pl.kernel,
        out_type=jax.ShapeDtypeStruct((NW, N_PAD), jnp.float32),
        mesh=_sc_mesh(),
        scratch_types=[
            pltpu.VMEM((nch, CHUNK), jnp.int32),
            pltpu.VMEM((N_PAD,), jnp.float32),
        ],
        **_SC_PARAMS,
    )
    def k(zeros_hbm, dst_hbm, out_hbm, dst_v, hist):
        c = lax.axis_index("c")
        s = lax.axis_index("s")
        wid = s * NC + c
        pltpu.sync_copy(dst_hbm.at[pl.ds(wid * nch, nch)], dst_v)
        pltpu.sync_copy(zeros_hbm, hist)
        ones = jnp.ones((16,), jnp.float32)

        def chunk(j, carry):
            for kk in range(CHUNK // 16):
                idx = dst_v[j, pl.ds(kk * 16, 16)]
                plsc.addupdate_scatter(hist, [idx], ones)
            return carry

        lax.fori_loop(0, nch, chunk, 0)
        pltpu.sync_copy(hist, out_hbm.at[wid])

    return k


def _make_agg_kernel(d, nch0, nch1):
    assert nch0 % (4 * ABUF) == 0 and nch1 % (4 * ABUF) == 0

    @functools.partial(
        pl.kernel,
        out_type=jax.ShapeDtypeStruct((NC * N_PAD, d), jnp.float32),
        mesh=_sc_mesh(),
        scratch_types=[
            pltpu.VMEM((2, ABUF, CHUNK), jnp.int32),
            pltpu.VMEM((4, ABUF, CHUNK), jnp.int32),
            pltpu.VMEM((2, ABUF, CHUNK, d), jnp.float32),
            pltpu.VMEM_SHARED((N_PAD, d), jnp.float32),
            pltpu.SemaphoreType.DMA,
            pltpu.SemaphoreType.DMA,
            pltpu.SemaphoreType.DMA,
            pltpu.SemaphoreType.DMA,
        ],
        **_SC_PARAMS,
    )
    def k(table_hbm, src_hbm, dst_hbm, out_hbm,
          srcb, dstb, rows, acc, sem_i0, sem_i1, sem_g, sem_s):
        c = lax.axis_index("c")
        s = lax.axis_index("s")
        rbase = s * RPT
        off = jnp.where(c == 0, s * nch0, NS * nch0 + s * nch1)
        ngroups = jnp.where(c == 0, nch0 // ABUF, nch1 // ABUF)

        def idx_start(j, sslot, dslot, sem):
            pltpu.async_copy(src_hbm.at[pl.ds(off + j * ABUF, ABUF)],
                             srcb.at[sslot], sem)
            pltpu.async_copy(dst_hbm.at[pl.ds(off + j * ABUF, ABUF)],
                             dstb.at[dslot], sem)

        def idx_wait(sem):
            for _ in range(2):
                pltpu.make_async_copy(src_hbm.at[pl.ds(0, ABUF)],
                                      srcb.at[0], sem).wait()

        def gather_start(p):
            for b in range(ABUF):
                pltpu.async_copy(table_hbm.at[srcb.at[p].at[b]],
                                 rows.at[p].at[b], sem_g)

        def gather_wait():
            for b in range(ABUF):
                pltpu.make_async_copy(table_hbm.at[pl.ds(0, CHUNK)],
                                      rows.at[0].at[b], sem_g).wait()

        def scatter_start(p, dslot):
            for b in range(ABUF):
                pltpu.async_copy(rows.at[p].at[b],
                                 acc.at[dstb.at[dslot].at[b]], sem_s, add=True)

        def scatter_wait():
            for b in range(ABUF):
                pltpu.make_async_copy(rows.at[0].at[b],
                                      acc.at[pl.ds(0, CHUNK)], sem_s).wait()

        zv = jnp.zeros((16,), jnp.float32)

        def zrow(i, carry):
            for kk in range(d // 16):
                rows[0, 0, i, pl.ds(kk * 16, 16)] = zv
            return carry

        lax.fori_loop(0, CHUNK, zrow, 0)
        for r5 in range(RPT // CHUNK):
            pltpu.sync_copy(rows.at[0].at[0],
                            acc.at[pl.ds(rbase + r5 * CHUNK, CHUNK)])

        plsc.subcore_barrier()

        idx_start(0, 0, 0, sem_i0)
        idx_start(1, 1, 1, sem_i1)
        idx_wait(sem_i0)
        gather_start(0)

        def quad(u, carry):
            for q in range(4):
                g = 4 * u + q
                p = q % 2
                sem_p = sem_i0 if p == 0 else sem_i1
                sem_o = sem_i1 if p == 0 else sem_i0
                gather_wait()
                if q == 0:
                    @pl.when(g > 0)
                    def _():
                        scatter_wait()
                else:
                    scatter_wait()

                @pl.when(g + 2 < ngroups)
                def _(sem_p=sem_p, g=g, p=p, q=q):
                    idx_start(g + 2, p, (q + 2) % 4, sem_p)

                scatter_start(p, q)

                @pl.when(g + 1 < ngroups)
                def _(sem_o=sem_o, p=p):
                    idx_wait(sem_o)
                    gather_start(1 - p)

            return carry

        lax.fori_loop(0, ngroups // 4, quad, 0)

        @pl.when(ngroups > 0)
        def _():
            scatter_wait()

        @pl.when(ngroups == 0)
        def _():
            gather_wait()
            idx_wait(sem_i1)

        plsc.subcore_barrier()
        obase = c * N_PAD + rbase
        for r5 in range(RPT // CHUNK):
            if r5 >= 2:
                pltpu.make_async_copy(rows.at[0].at[0],
                                      out_hbm.at[pl.ds(0, CHUNK)],
                                      sem_g).wait()
            pltpu.sync_copy(acc.at[pl.ds(rbase + r5 * CHUNK, CHUNK)],
                            rows.at[r5 % 2].at[0])
            pltpu.async_copy(rows.at[r5 % 2].at[0],
                             out_hbm.at[pl.ds(obase + r5 * CHUNK, CHUNK)],
                             sem_g)
        for _ in range(2):
            pltpu.make_async_copy(rows.at[0].at[0],
                                  out_hbm.at[pl.ds(0, CHUNK)],
                                  sem_g).wait()

    return k


_BM = 1024
_GRID = (N_PAD // _BM,)


def _tc_h_body(x_ref, w_ref, o_ref):
    o_ref[...] = jnp.dot(x_ref[...], w_ref[...],
                         preferred_element_type=jnp.float32)


def _tc_scale_body(h_ref, dp_ref, o_ref, dv_ref):
    deg = jnp.sum(dp_ref[...], axis=0, keepdims=True) + 1.0
    dinv = jnp.transpose(lax.rsqrt(deg))
    o_ref[...] = h_ref[...] * dinv
    dv_ref[...] = jnp.broadcast_to(dinv, (dinv.shape[0], 16))


def _tc_mid_body(a0_ref, a1_ref, h2_ref, dv_ref, b1_ref, w_ref, o_ref):
    dinv = dv_ref[:, 0:1]
    a = (a0_ref[...] + a1_ref[...] + h2_ref[...]) * dinv + b1_ref[...]
    hr = jnp.maximum(a, 0.0)
    g = jnp.dot(hr, w_ref[...], preferred_element_type=jnp.float32)
    o_ref[...] = g * dinv


def _tc_out_body(a0_ref, a1_ref, g2_ref, dv_ref, b2_ref, ls_ref, lg_ref):
    dinv = dv_ref[:, 0:1]
    logits = (a0_ref[...] + a1_ref[...] + g2_ref[...]) * dinv + b2_ref[...]
    col = lax.broadcasted_iota(jnp.int32, (_BM, C_PAD), 1)
    valid = col < N_CLASSES
    m = jnp.max(jnp.where(valid, logits, -1e30), axis=1, keepdims=True)
    e = jnp.where(valid, jnp.exp(logits - m), 0.0)
    ssum = jnp.sum(e, axis=1, keepdims=True)
    ls_ref[...] = logits - m - jnp.log(ssum)
    lg_ref[...] = logits


def kernel(x, edge_index, W1, b1, W2, b2):
    src = edge_index[0].astype(jnp.int32)
    dst = edge_index[1].astype(jnp.int32)
    n_edges = src.shape[0]
    per_s = -(-n_edges // (NS * CHUNK))
    per_s = -(-per_s // 16) * 16
    e_pad = NS * per_s * CHUNK
    npad_e = e_pad - n_edges
    ar = jnp.arange(npad_e, dtype=jnp.int32)
    pad_src = ar % N_NODES
    pad_dst = N_NODES + ar % (N_PAD - N_NODES)
    src2 = jnp.concatenate([src, pad_src]).reshape(NS * per_s, CHUNK)
    dst2 = jnp.concatenate([dst, pad_dst]).reshape(NS * per_s, CHUNK)

    xp = jnp.zeros((N_PAD, D_FEAT), jnp.float32).at[:N_NODES].set(x)
    zeros1 = jnp.zeros((N_PAD,), jnp.float32)
    W2p = jnp.zeros((HIDDEN, C_PAD), jnp.float32).at[:, :N_CLASSES].set(W2)
    b1r = b1.reshape(1, HIDDEN)
    b2r = jnp.zeros((1, C_PAD), jnp.float32).at[0, :N_CLASSES].set(b2)

    degp = _make_deg_kernel(per_s)(zeros1, dst2)

    h = pl.pallas_call(
        _tc_h_body,
        grid=_GRID,
        in_specs=[
            pl.BlockSpec((_BM, D_FEAT), lambda i: (i, 0)),
            pl.BlockSpec((D_FEAT, HIDDEN), lambda i: (0, 0)),
        ],
        out_specs=pl.BlockSpec((_BM, HIDDEN), lambda i: (i, 0)),
        out_shape=jax.ShapeDtypeStruct((N_PAD, HIDDEN), jnp.float32),
    )(xp, W1)

    h2, dinv16 = pl.pallas_call(
        _tc_scale_body,
        grid=_GRID,
        in_specs=[
            pl.BlockSpec((_BM, HIDDEN), lambda i: (i, 0)),
            pl.BlockSpec((NW, _BM), lambda i: (0, i)),
        ],
        out_specs=[
            pl.BlockSpec((_BM, HIDDEN), lambda i: (i, 0)),
            pl.BlockSpec((_BM, 16), lambda i: (i, 0)),
        ],
        out_shape=[
            jax.ShapeDtypeStruct((N_PAD, HIDDEN), jnp.float32),
            jax.ShapeDtypeStruct((N_PAD, 16), jnp.float32),
        ],
    )(h, degp)

    a0, a1 = _split(per_s, 0.5)
    agg1 = _make_agg_kernel(HIDDEN, a0, a1)(h2, src2, dst2)

    g2 = pl.pallas_call(
        _tc_mid_body,
        grid=_GRID,
        in_specs=[
            pl.BlockSpec((_BM, HIDDEN), lambda i: (i, 0)),
            pl.BlockSpec((_BM, HIDDEN), lambda i: (N_PAD // _BM + i, 0)),
            pl.BlockSpec((_BM, HIDDEN), lambda i: (i, 0)),
            pl.BlockSpec((_BM, 16), lambda i: (i, 0)),
            pl.BlockSpec((1, HIDDEN), lambda i: (0, 0)),
            pl.BlockSpec((HIDDEN, C_PAD), lambda i: (0, 0)),
        ],
        out_specs=pl.BlockSpec((_BM, C_PAD), lambda i: (i, 0)),
        out_shape=jax.ShapeDtypeStruct((N_PAD, C_PAD), jnp.float32),
    )(agg1, agg1, h2, dinv16, b1r, W2p)

    agg2 = _make_agg_kernel(C_PAD, a0, a1)(g2, src2, dst2)

    ls, lg = pl.pallas_call(
        _tc_out_body,
        grid=_GRID,
        in_specs=[
            pl.BlockSpec((_BM, C_PAD), lambda i: (i, 0)),
            pl.BlockSpec((_BM, C_PAD), lambda i: (N_PAD // _BM + i, 0)),
            pl.BlockSpec((_BM, C_PAD), lambda i: (i, 0)),
            pl.BlockSpec((_BM, 16), lambda i: (i, 0)),
            pl.BlockSpec((1, C_PAD), lambda i: (0, 0)),
        ],
        out_specs=[
            pl.BlockSpec((_BM, C_PAD), lambda i: (i, 0)),
            pl.BlockSpec((_BM, C_PAD), lambda i: (i, 0)),
        ],
        out_shape=[
            jax.ShapeDtypeStruct((N_PAD, C_PAD), jnp.float32),
            jax.ShapeDtypeStruct((N_PAD, C_PAD), jnp.float32),
        ],
    )(agg2, agg2, g2, dinv16, b2r)

    return (ls[:N_NODES, :N_CLASSES], lg[:N_NODES, :N_CLASSES])
